# baseline (speedup 1.0000x reference)
import functools

import jax
import jax.numpy as jnp
from jax import lax
from jax.experimental import pallas as pl
from jax.experimental.pallas import tpu as pltpu
from jax.experimental.pallas import tpu_sc as plsc

V = 1000
VP = 1024
N_TOK = 1024 * 50
NC, NS, LANES = 2, 16, 16
NW = NC * NS
NB = 16
BC = 64
NG = NW // NB
TB = 256
TOK_PER_G = N_TOK // NG
N_ITEM = TOK_PER_G // TB
G1 = (V - 15 * BC) // 8
G2 = BC // 8 - G1


def _lse_body(tab_ref, lse_ref):
    t = tab_ref[...]
    m = jnp.max(t, axis=1)
    lse_ref[...] = m + jnp.log(jnp.sum(jnp.exp(t - m[:, None]), axis=1))


def _row_lse(table):
    return pl.pallas_call(
        _lse_body,
        out_shape=jax.ShapeDtypeStruct((V,), jnp.float32),
    )(table)


_MESH = plsc.VectorSubcoreMesh(core_axis_name="c", subcore_axis_name="s")


@functools.partial(
    pl.kernel,
    out_type=[
        jax.ShapeDtypeStruct((V // 8, 8, N_TOK), jnp.float32),
        jax.ShapeDtypeStruct((NW, LANES), jnp.float32),
    ],
    mesh=_MESH,
    compiler_params=pltpu.CompilerParams(use_tc_tiling_on_sc=True,
                                         needs_layout_passes=False),
    scratch_types=[
        pltpu.VMEM((BC, 8, 128), jnp.float32),
        pltpu.VMEM((V,), jnp.float32),
        [pltpu.VMEM((TB,), jnp.int32)] * 2,
        [pltpu.VMEM((TB,), jnp.int32)] * 2,
        [pltpu.VMEM((BC // 8, 8, TB), jnp.float32)] * 2,
        pltpu.VMEM((LANES,), jnp.float32),
        [pltpu.SemaphoreType.DMA] * 2,
        [pltpu.SemaphoreType.DMA] * 2,
    ],
)
def _sc_main(tabt_hbm, idx_hbm, tgt_hbm, lse_hbm, xt_hbm, part_hbm,
             tab_v, lse_v, idx_r, tgt_r, stg, part_v, sem_f, sem_s):
    wid = lax.axis_index("s") * NC + lax.axis_index("c")
    q = wid % NB
    tok0 = (wid // NB) * TOK_PER_G
    gq = q * (BC // 8)
    pltpu.sync_copy(tabt_hbm.at[pl.ds(q * BC, BC)], tab_v)
    pltpu.sync_copy(lse_hbm, lse_v)

    def start_fetch(m, b):
        s = pl.ds(tok0 + m * TB, TB)
        pltpu.make_async_copy(idx_hbm.at[s], idx_r[b], sem_f[b]).start()
        pltpu.make_async_copy(tgt_hbm.at[s], tgt_r[b], sem_f[b]).start()

    def wait_fetch(b):
        s = pl.ds(tok0, TB)
        pltpu.make_async_copy(idx_hbm.at[s], idx_r[b], sem_f[b]).wait()
        pltpu.make_async_copy(tgt_hbm.at[s], tgt_r[b], sem_f[b]).wait()

    def start_scatter(m, b):
        t0 = tok0 + m * TB
        pltpu.make_async_copy(
            stg[b].at[pl.ds(0, G1)],
            xt_hbm.at[pl.ds(gq, G1), slice(None), pl.ds(t0, TB)],
            sem_s[b]).start()

        @pl.when(q < NB - 1)
        def _():
            pltpu.make_async_copy(
                stg[b].at[pl.ds(G1, G2)],
                xt_hbm.at[pl.ds(gq + G1, G2), slice(None), pl.ds(t0, TB)],
                sem_s[b]).start()

    def wait_scatter(b):
        pltpu.make_async_copy(
            stg[b].at[pl.ds(0, G1)],
            xt_hbm.at[pl.ds(gq, G1), slice(None), pl.ds(0, TB)],
            sem_s[b]).wait()

        @pl.when(q < NB - 1)
        def _():
            pltpu.make_async_copy(
                stg[b].at[pl.ds(G1, G2)],
                xt_hbm.at[pl.ds(gq + G1, G2), slice(None), pl.ds(0, TB)],
                sem_s[b]).wait()

    start_fetch(0, 0)
    start_fetch(1, 1)
    q16 = jnp.full((LANES,), q, jnp.int32)

    def item(m, acc):
        for b in range(2):
            k = m * 2 + b
            wait_fetch(b)

            @pl.when(k >= 2)
            def _():
                wait_scatter(b)

            rt = []
            rl = []
            for tt in range(TB // LANES):
                iv = idx_r[b][pl.ds(tt * LANES, LANES)]
                rt.append(jnp.right_shift(iv, 7))
                rl.append(jnp.bitwise_and(iv, 127))

            def col(cl, c):
                cf = jnp.full((LANES,), cl, jnp.int32)
                gc = jnp.right_shift(cl, 3)
                sc = jnp.bitwise_and(cl, 7)
                for tt in range(TB // LANES):
                    v = plsc.load_gather(tab_v, [cf, rt[tt], rl[tt]])
                    stg[b][gc, sc, pl.ds(tt * LANES, LANES)] = v
                return c

            lax.fori_loop(0, BC, col, 0)

            for tt in range(TB // LANES):
                s = pl.ds(tt * LANES, LANES)
                tg = tgt_r[b][s]
                picked = plsc.load_gather(
                    tab_v, [jnp.bitwise_and(tg, BC - 1), rt[tt], rl[tt]])
                lseg = plsc.load_gather(
                    lse_v, [rt[tt] * 128 + rl[tt]])
                hit = jnp.right_shift(tg, 6) == q16
                acc = acc + jnp.where(hit, lseg - picked,
                                      jnp.zeros((LANES,), jnp.float32))

            start_scatter(k, b)

            @pl.when(k + 2 < N_ITEM)
            def _():
                start_fetch(k + 2, b)
        return acc

    acc = lax.fori_loop(0, N_ITEM // 2, item,
                        jnp.zeros((LANES,), jnp.float32))
    for b in range(2):
        wait_scatter(b)
    part_v[...] = acc
    pltpu.sync_copy(part_v, part_hbm.at[wid])


def _loss_body(part_ref, out_ref):
    out_ref[0, 0] = jnp.sum(part_ref[...]) * (1.0 / N_TOK)


def _final_loss(partials):
    return pl.pallas_call(
        _loss_body,
        out_shape=jax.ShapeDtypeStruct((1, 1), jnp.float32),
        out_specs=pl.BlockSpec(memory_space=pltpu.SMEM),
    )(partials)


def kernel(input_index, targets, token_embedding_table):
    idx = input_index.reshape(-1).astype(jnp.int32)
    tgt = targets.reshape(-1).astype(jnp.int32)
    table = token_embedding_table
    lse = _row_lse(table)
    tabt = jnp.pad(table.T, ((0, 0), (0, VP - V))).reshape(V, 8, 128)
    xt, partials = _sc_main(tabt, idx, tgt, lse)
    loss = _final_loss(partials)[0, 0]
    return (xt.reshape(V, N_TOK).T, loss)


# speedup vs baseline: 3.0884x; 3.0884x over previous
import functools

import jax
import jax.numpy as jnp
from jax import lax
from jax.experimental import pallas as pl
from jax.experimental.pallas import tpu as pltpu
from jax.experimental.pallas import tpu_sc as plsc

V = 1000
VP = 1024
N_TOK = 1024 * 50
NC, NS, LANES = 2, 16, 16
NW = NC * NS
NB = 16
BC = 64
NG = NW // NB
TB = 256
TOK_PER_G = N_TOK // NG
N_ITEM = TOK_PER_G // TB
G1 = (V - 15 * BC) // 8
G2 = BC // 8 - G1


def _lse_body(tab_ref, lse_ref):
    t = tab_ref[...]
    m = jnp.max(t, axis=1)
    lse_ref[...] = m + jnp.log(jnp.sum(jnp.exp(t - m[:, None]), axis=1))


def _row_lse(table):
    return pl.pallas_call(
        _lse_body,
        out_shape=jax.ShapeDtypeStruct((V,), jnp.float32),
    )(table)


_MESH = plsc.VectorSubcoreMesh(core_axis_name="c", subcore_axis_name="s")


@functools.partial(
    pl.kernel,
    out_type=[
        jax.ShapeDtypeStruct((V // 8, 8, N_TOK), jnp.float32),
        jax.ShapeDtypeStruct((NW, LANES), jnp.float32),
    ],
    mesh=_MESH,
    compiler_params=pltpu.CompilerParams(use_tc_tiling_on_sc=True,
                                         needs_layout_passes=False),
    scratch_types=[
        pltpu.VMEM((BC * VP,), jnp.float32),
        pltpu.VMEM((V,), jnp.float32),
        [pltpu.VMEM((TB,), jnp.int32)] * 2,
        [pltpu.VMEM((TB,), jnp.int32)] * 2,
        [pltpu.VMEM((BC // 8, 8, TB), jnp.float32)] * 2,
        pltpu.VMEM((LANES,), jnp.float32),
        [pltpu.SemaphoreType.DMA] * 2,
        [pltpu.SemaphoreType.DMA] * 2,
    ],
)
def _sc_main(tabt_hbm, idx_hbm, tgt_hbm, lse_hbm, xt_hbm, part_hbm,
             tab_v, lse_v, idx_r, tgt_r, stg, part_v, sem_f, sem_s):
    wid = lax.axis_index("s") * NC + lax.axis_index("c")
    q = wid % NB
    tok0 = (wid // NB) * TOK_PER_G
    gq = q * (BC // 8)
    pltpu.sync_copy(tabt_hbm.at[pl.ds(q * BC * VP, BC * VP)], tab_v)
    pltpu.sync_copy(lse_hbm, lse_v)

    def start_fetch(m, b):
        s = pl.ds(tok0 + m * TB, TB)
        pltpu.make_async_copy(idx_hbm.at[s], idx_r[b], sem_f[b]).start()
        pltpu.make_async_copy(tgt_hbm.at[s], tgt_r[b], sem_f[b]).start()

    def wait_fetch(b):
        s = pl.ds(tok0, TB)
        pltpu.make_async_copy(idx_hbm.at[s], idx_r[b], sem_f[b]).wait()
        pltpu.make_async_copy(tgt_hbm.at[s], tgt_r[b], sem_f[b]).wait()

    def start_scatter(m, b):
        t0 = tok0 + m * TB
        pltpu.make_async_copy(
            stg[b].at[pl.ds(0, G1)],
            xt_hbm.at[pl.ds(gq, G1), slice(None), pl.ds(t0, TB)],
            sem_s[b]).start()

        @pl.when(q < NB - 1)
        def _():
            pltpu.make_async_copy(
                stg[b].at[pl.ds(G1, G2)],
                xt_hbm.at[pl.ds(gq + G1, G2), slice(None), pl.ds(t0, TB)],
                sem_s[b]).start()

    def wait_scatter(b):
        pltpu.make_async_copy(
            stg[b].at[pl.ds(0, G1)],
            xt_hbm.at[pl.ds(gq, G1), slice(None), pl.ds(0, TB)],
            sem_s[b]).wait()

        @pl.when(q < NB - 1)
        def _():
            pltpu.make_async_copy(
                stg[b].at[pl.ds(G1, G2)],
                xt_hbm.at[pl.ds(gq + G1, G2), slice(None), pl.ds(0, TB)],
                sem_s[b]).wait()

    start_fetch(0, 0)
    start_fetch(1, 1)
    q16 = jnp.full((LANES,), q, jnp.int32)

    def item(m, acc):
        for b in range(2):
            k = m * 2 + b
            wait_fetch(b)

            @pl.when(k >= 2)
            def _():
                wait_scatter(b)

            iv = [idx_r[b][pl.ds(tt * LANES, LANES)]
                  for tt in range(TB // LANES)]

            @plsc.parallel_loop(0, BC, 1, unroll=4)
            def col(cl):
                base = cl * VP
                gc = jnp.right_shift(cl, 3)
                sc = jnp.bitwise_and(cl, 7)
                for tt in range(TB // LANES):
                    v = plsc.load_gather(tab_v, [iv[tt] + base])
                    stg[b][gc, sc, pl.ds(tt * LANES, LANES)] = v

            for tt in range(TB // LANES):
                s = pl.ds(tt * LANES, LANES)
                tg = tgt_r[b][s]
                picked = plsc.load_gather(
                    tab_v, [iv[tt] + jnp.bitwise_and(tg, BC - 1) * VP])
                lseg = plsc.load_gather(lse_v, [iv[tt]])
                hit = jnp.right_shift(tg, 6) == q16
                acc = acc + jnp.where(hit, lseg - picked,
                                      jnp.zeros((LANES,), jnp.float32))

            start_scatter(k, b)

            @pl.when(k + 2 < N_ITEM)
            def _():
                start_fetch(k + 2, b)
        return acc

    acc = lax.fori_loop(0, N_ITEM // 2, item,
                        jnp.zeros((LANES,), jnp.float32))
    for b in range(2):
        wait_scatter(b)
    part_v[...] = acc
    pltpu.sync_copy(part_v, part_hbm.at[wid])


def _loss_body(part_ref, out_ref):
    out_ref[0, 0] = jnp.sum(part_ref[...]) * (1.0 / N_TOK)


def _final_loss(partials):
    return pl.pallas_call(
        _loss_body,
        out_shape=jax.ShapeDtypeStruct((1, 1), jnp.float32),
        out_specs=pl.BlockSpec(memory_space=pltpu.SMEM),
    )(partials)


def kernel(input_index, targets, token_embedding_table):
    idx = input_index.reshape(-1).astype(jnp.int32)
    tgt = targets.reshape(-1).astype(jnp.int32)
    table = token_embedding_table
    lse = _row_lse(table)
    tabt = jnp.pad(table.T, ((0, 0), (0, VP - V))).reshape(-1)
    xt, partials = _sc_main(tabt, idx, tgt, lse)
    loss = _final_loss(partials)[0, 0]
    return (xt.reshape(V, N_TOK).T, loss)
